# Initial kernel scaffold; baseline (speedup 1.0000x reference)
import jax
import jax.numpy as jnp
from jax.experimental import pallas as pl


def kernel(x, edge_index, batch, W1, att_src1, att_dst1, b1, W2, att_src2, att_dst2, b2, W3, att_src3, att_dst3, b3, ln_w1, ln_b1, ln_w2, ln_b2, ln_w3, ln_b3):
    raise NotImplementedError("write your pallas kernel here")


# speedup vs baseline: 4.3456x; 4.3456x over previous
import jax
import jax.numpy as jnp
import numpy as np
from jax.experimental import pallas as pl

NUM_GRAPHS = 2048

_BN = 2048
_BE = 4096
_BM = 2048


def _proj_kernel(x_ref, w_ref, asrc_ref, adst_ref, h_ref, as_ref, ad_ref):
    h = jnp.dot(x_ref[...], w_ref[...], preferred_element_type=jnp.float32)
    h_ref[...] = h
    as_ref[...] = jnp.dot(h, asrc_ref[...], preferred_element_type=jnp.float32)
    ad_ref[...] = jnp.dot(h, adst_ref[...], preferred_element_type=jnp.float32)


def _alpha_kernel(as_ref, ad_ref, out_ref):
    a = as_ref[...] + ad_ref[...]
    out_ref[...] = jnp.where(a > 0, a, 0.2 * a)


def _ea_kernel(alpha_ref, amax_ref, out_ref):
    out_ref[...] = jnp.exp(alpha_ref[...] - amax_ref[...])


def _msg_kernel(h_ref, ea_ref, den_ref, p_ref, out_ref):
    coef = ea_ref[...] / jnp.maximum(den_ref[...], 1e-16)
    ce = jnp.dot(coef, p_ref[...], preferred_element_type=jnp.float32)
    out_ref[...] = h_ref[...] * ce


def _ln_kernel(x_ref, mean_ref, rstd_ref, w_ref, b_ref, out_ref):
    y = (x_ref[...] - mean_ref[...]) * rstd_ref[...] * w_ref[...] + b_ref[...]
    out_ref[...] = jnp.maximum(y, 0.0)


def _proj(x, W, att_src, att_dst):
    n, din = x.shape
    heads, C = att_src.shape[1], att_src.shape[2]
    dout = heads * C
    mask = (np.arange(dout)[:, None] // C) == np.arange(heads)[None, :]
    A_src = jnp.where(mask, att_src.reshape(dout)[:, None], 0.0)
    A_dst = jnp.where(mask, att_dst.reshape(dout)[:, None], 0.0)
    return pl.pallas_call(
        _proj_kernel,
        grid=(pl.cdiv(n, _BN),),
        in_specs=[
            pl.BlockSpec((_BN, din), lambda i: (i, 0)),
            pl.BlockSpec((din, dout), lambda i: (0, 0)),
            pl.BlockSpec((dout, heads), lambda i: (0, 0)),
            pl.BlockSpec((dout, heads), lambda i: (0, 0)),
        ],
        out_specs=[
            pl.BlockSpec((_BN, dout), lambda i: (i, 0)),
            pl.BlockSpec((_BN, heads), lambda i: (i, 0)),
            pl.BlockSpec((_BN, heads), lambda i: (i, 0)),
        ],
        out_shape=[
            jax.ShapeDtypeStruct((n, dout), jnp.float32),
            jax.ShapeDtypeStruct((n, heads), jnp.float32),
            jax.ShapeDtypeStruct((n, heads), jnp.float32),
        ],
    )(x, W, A_src, A_dst)


def _edge_ew(body, a, b):
    e, heads = a.shape
    return pl.pallas_call(
        body,
        grid=(pl.cdiv(e, _BE),),
        in_specs=[
            pl.BlockSpec((_BE, heads), lambda i: (i, 0)),
            pl.BlockSpec((_BE, heads), lambda i: (i, 0)),
        ],
        out_specs=pl.BlockSpec((_BE, heads), lambda i: (i, 0)),
        out_shape=jax.ShapeDtypeStruct((e, heads), jnp.float32),
    )(a, b)


def _msg(hsrc, ea, den, heads, C):
    e, dout = hsrc.shape
    mask = (np.arange(dout)[None, :] // C) == np.arange(heads)[:, None]
    P = jnp.asarray(mask, jnp.float32)
    return pl.pallas_call(
        _msg_kernel,
        grid=(pl.cdiv(e, _BM),),
        in_specs=[
            pl.BlockSpec((_BM, dout), lambda i: (i, 0)),
            pl.BlockSpec((_BM, heads), lambda i: (i, 0)),
            pl.BlockSpec((_BM, heads), lambda i: (i, 0)),
            pl.BlockSpec((heads, dout), lambda i: (0, 0)),
        ],
        out_specs=pl.BlockSpec((_BM, dout), lambda i: (i, 0)),
        out_shape=jax.ShapeDtypeStruct((e, dout), jnp.float32),
    )(hsrc, ea, den, P)


def _gat_layer(x, src, dst, W, att_src, att_dst, bias, heads, C):
    n = x.shape[0]
    h, a_s, a_d = _proj(x, W, att_src, att_dst)
    alpha = _edge_ew(_alpha_kernel, a_s[src], a_d[dst])
    amax = jax.ops.segment_max(alpha, dst, num_segments=n)
    amax = jnp.where(jnp.isfinite(amax), amax, 0.0)
    ea = _edge_ew(_ea_kernel, alpha, amax[dst])
    denom = jax.ops.segment_sum(ea, dst, num_segments=n)
    msg = _msg(h[src], ea, denom[dst], heads, C)
    out = jax.ops.segment_sum(msg, dst, num_segments=n)
    return out + bias


def _ln_relu(x, batch, w, b, counts):
    n, Cf = x.shape
    norm = jnp.maximum(counts, 1.0) * Cf
    mean = jax.ops.segment_sum(x.sum(axis=-1), batch, num_segments=NUM_GRAPHS) / norm
    sumsq = jax.ops.segment_sum(
        jnp.sum(x * x, axis=-1), batch, num_segments=NUM_GRAPHS) / norm
    var = sumsq - mean * mean
    rstd = 1.0 / jnp.sqrt(var + 1e-5)
    mean_g = mean[batch][:, None]
    rstd_g = rstd[batch][:, None]
    return pl.pallas_call(
        _ln_kernel,
        grid=(pl.cdiv(n, _BN),),
        in_specs=[
            pl.BlockSpec((_BN, Cf), lambda i: (i, 0)),
            pl.BlockSpec((_BN, 1), lambda i: (i, 0)),
            pl.BlockSpec((_BN, 1), lambda i: (i, 0)),
            pl.BlockSpec((1, Cf), lambda i: (0, 0)),
            pl.BlockSpec((1, Cf), lambda i: (0, 0)),
        ],
        out_specs=pl.BlockSpec((_BN, Cf), lambda i: (i, 0)),
        out_shape=jax.ShapeDtypeStruct((n, Cf), jnp.float32),
    )(x, mean_g, rstd_g, w[None, :], b[None, :])


def kernel(x, edge_index, batch, W1, att_src1, att_dst1, b1, W2, att_src2,
           att_dst2, b2, W3, att_src3, att_dst3, b3, ln_w1, ln_b1, ln_w2,
           ln_b2, ln_w3, ln_b3):
    n = x.shape[0]
    loop = jnp.arange(n, dtype=edge_index.dtype)
    src = jnp.concatenate([edge_index[0], loop])
    dst = jnp.concatenate([edge_index[1], loop])

    ones = jnp.ones((n,), jnp.float32)
    counts = jax.ops.segment_sum(ones, batch, num_segments=NUM_GRAPHS)

    x1 = _gat_layer(x, src, dst, W1, att_src1, att_dst1, b1, 4, 8)
    x1 = _ln_relu(x1, batch, ln_w1, ln_b1, counts)
    x2 = _gat_layer(x1, src, dst, W2, att_src2, att_dst2, b2, 4, 16)
    x2 = _ln_relu(x2, batch, ln_w2, ln_b2, counts)
    x3 = _gat_layer(x2, src, dst, W3, att_src3, att_dst3, b3, 1, 128)
    x3 = _ln_relu(x3, batch, ln_w3, ln_b3, counts)

    s = jax.ops.segment_sum(x3, batch, num_segments=NUM_GRAPHS)
    return s / jnp.maximum(counts, 1.0)[:, None]
